# baseline (speedup 1.0000x reference)
import functools

import jax
import jax.numpy as jnp
from jax import lax
from jax.experimental import pallas as pl
from jax.experimental.pallas import tpu as pltpu
from jax.experimental.pallas import tpu_sc as plsc

VOCAB = 1000
CTX = 50
BATCH = 1024
DIM = VOCAB + CTX

_NC, _NS, _L = 2, 16, 16
_NW = _NC * _NS
_BPW = BATCH // _NW
_NQ = (CTX + _L - 1) // _L


def _slab_body(tok_hbm, zero_hbm, out_ref, tok_v, slab_v, sems):
    wid = lax.axis_index("s") * _NC + lax.axis_index("c")
    b0 = wid * _BPW

    pltpu.sync_copy(
        tok_hbm.at[pl.ds(b0 * CTX, _BPW * CTX)], tok_v.at[pl.ds(0, _BPW * CTX)]
    )
    pltpu.sync_copy(zero_hbm, slab_v.at[0])
    pltpu.sync_copy(zero_hbm, slab_v.at[1])

    iota = lax.iota(jnp.int32, _L)
    ones16 = jnp.full((_L,), 1.0, jnp.float32)
    zeros16 = jnp.full((_L,), 0.0, jnp.float32)

    def put(bi, slot, val16):
        for q in range(_NQ):
            r = q * _L + iota
            m = r < CTX
            tok = tok_v[pl.ds(bi * CTX + q * _L, _L)]
            plsc.store_scatter(slab_v.at[slot], [r, tok], val16, mask=m)
            plsc.store_scatter(slab_v.at[slot], [r, VOCAB + r], val16, mask=m)

    def step(bi2, carry):
        for s in range(2):
            bi = bi2 * 2 + s

            @pl.when(bi >= 2)
            def _reclaim(bi=bi, s=s):
                pltpu.make_async_copy(
                    slab_v.at[s], out_ref.at[b0 + bi - 2], sems.at[s]
                ).wait()
                put(bi - 2, s, zeros16)

            put(bi, s, ones16)
            pltpu.async_copy(slab_v.at[s], out_ref.at[b0 + bi], sems.at[s])
        return carry

    lax.fori_loop(0, _BPW // 2, step, None)

    for bi in (_BPW - 2, _BPW - 1):
        s = bi % 2
        pltpu.make_async_copy(
            slab_v.at[s], out_ref.at[b0 + bi], sems.at[s]
        ).wait()


@functools.partial(
    pl.kernel,
    out_type=jax.ShapeDtypeStruct((BATCH, CTX, DIM), jnp.float32),
    mesh=plsc.VectorSubcoreMesh(core_axis_name="c", subcore_axis_name="s"),
    compiler_params=pltpu.CompilerParams(needs_layout_passes=False),
    scratch_types=[
        pltpu.VMEM((_BPW * CTX + _L * _NQ,), jnp.int32),
        pltpu.VMEM((2, CTX, DIM), jnp.float32),
        pltpu.SemaphoreType.DMA((2,)),
    ],
)
def _build_onehot(tok_hbm, zero_hbm, out_ref, tok_v, slab_v, sems):
    _slab_body(tok_hbm, zero_hbm, out_ref, tok_v, slab_v, sems)


def kernel(token_ids):
    tok_flat = token_ids.reshape(-1).astype(jnp.int32)
    zero_slab = jnp.zeros((CTX, DIM), jnp.float32)
    return _build_onehot(tok_flat, zero_slab)


# speedup vs baseline: 1.0336x; 1.0336x over previous
import functools

import jax
import jax.numpy as jnp
from jax import lax
from jax.experimental import pallas as pl
from jax.experimental.pallas import tpu as pltpu
from jax.experimental.pallas import tpu_sc as plsc

VOCAB = 1000
CTX = 50
BATCH = 1024
DIM = VOCAB + CTX

_NC, _NS, _L = 2, 16, 16
_NW = _NC * _NS
_BPW = BATCH // _NW
_NTR = (CTX + 7) // 8


def _piece_rows(tr):
    return min(8, CTX - tr * 8)


def _slab_body(tok_hbm, zero_hbm, out_ref, tok_v, piece_v, sems):
    wid = lax.axis_index("s") * _NC + lax.axis_index("c")
    b0 = wid * _BPW

    pltpu.sync_copy(
        tok_hbm.at[pl.ds(b0 * CTX, _BPW * CTX)], tok_v.at[pl.ds(0, _BPW * CTX)]
    )
    for tr in range(_NTR):
        pltpu.sync_copy(zero_hbm, piece_v.at[tr])

    iota = lax.iota(jnp.int32, _L)
    rl = lax.rem(iota, 8)
    ones16 = jnp.full((_L,), 1.0, jnp.float32)
    zeros16 = jnp.full((_L,), 0.0, jnp.float32)

    def put(bi, tr, val16):
        r = tr * 8 + rl
        mask = r < CTX
        tok = plsc.load_gather(tok_v, [bi * CTX + r])
        col = jnp.where(iota < 8, tok, VOCAB + r)
        plsc.store_scatter(piece_v.at[tr], [rl, col], val16, mask=mask)

    def fire(bi, tr):
        rows = _piece_rows(tr)
        src = piece_v.at[tr] if rows == 8 else piece_v.at[tr, pl.ds(0, rows)]
        pltpu.async_copy(
            src, out_ref.at[b0 + bi, pl.ds(tr * 8, rows)], sems.at[tr]
        )

    def wait(bi, tr):
        rows = _piece_rows(tr)
        src = piece_v.at[tr] if rows == 8 else piece_v.at[tr, pl.ds(0, rows)]
        pltpu.make_async_copy(
            src, out_ref.at[b0 + bi, pl.ds(tr * 8, rows)], sems.at[tr]
        ).wait()

    for tr in range(_NTR):
        put(0, tr, ones16)
        fire(0, tr)

    def step(bi, carry):
        for tr in range(_NTR):
            wait(bi - 1, tr)
            put(bi - 1, tr, zeros16)
            put(bi, tr, ones16)
            fire(bi, tr)
        return carry

    lax.fori_loop(1, _BPW, step, None)

    for tr in range(_NTR):
        wait(_BPW - 1, tr)


@functools.partial(
    pl.kernel,
    out_type=jax.ShapeDtypeStruct((BATCH, CTX, DIM), jnp.float32),
    mesh=plsc.VectorSubcoreMesh(core_axis_name="c", subcore_axis_name="s"),
    compiler_params=pltpu.CompilerParams(needs_layout_passes=False),
    scratch_types=[
        pltpu.VMEM((_BPW * CTX + _L,), jnp.int32),
        pltpu.VMEM((_NTR, 8, DIM), jnp.float32),
        pltpu.SemaphoreType.DMA((_NTR,)),
    ],
)
def _build_onehot(tok_hbm, zero_hbm, out_ref, tok_v, piece_v, sems):
    _slab_body(tok_hbm, zero_hbm, out_ref, tok_v, piece_v, sems)


def kernel(token_ids):
    tok_flat = token_ids.reshape(-1).astype(jnp.int32)
    zero_piece = jnp.zeros((8, DIM), jnp.float32)
    return _build_onehot(tok_flat, zero_piece)
